# baseline (speedup 1.0000x reference)
import jax
import jax.numpy as jnp
from jax import lax
from jax.experimental import pallas as pl
from jax.experimental.pallas import tpu as pltpu
from jax.experimental.pallas import tpu_sc as plsc

VOCAB = 1000000
EMBED_DIM = 64
BATCH = 4096
HIST = 50

NC, NS = 2, 16
NW = NC * NS
B_CH = 8
CHUNK = B_CH * HIST
N_IDX = BATCH * HIST
CPW = N_IDX // (NW * CHUNK)
BPW = BATCH // NW

_mesh = plsc.VectorSubcoreMesh(core_axis_name="c", subcore_axis_name="s",
                               num_cores=NC, num_subcores=NS)


def _body(idx_hbm, tbl_hbm, out_hbm, idx_v, rows0, rows1, gsem0, gsem1,
          osem0, osem1):
    wid = lax.axis_index("s") * NC + lax.axis_index("c")
    bbase = wid * BPW
    pltpu.sync_copy(idx_hbm.at[wid], idx_v)

    rows = (rows0, rows1)
    gsem = (gsem0, gsem1)
    osem = (osem0, osem1)

    def gather(j, b):
        return pltpu.async_copy(tbl_hbm.at[idx_v.at[j]], rows[b], gsem[b])

    def outcopy(j, b):
        descs = []
        for k in range(B_CH):
            descs.append(pltpu.async_copy(
                rows[b].at[pl.ds(k * HIST, HIST)],
                out_hbm.at[bbase + j * B_CH + k], osem[b]))
        return descs

    def wait_all(descs):
        for d in descs:
            d.wait()

    g = [None, None]
    o = [None, None]
    g[0] = gather(0, 0)
    for j in range(CPW):
        b, nb = j % 2, (j + 1) % 2
        if j + 1 < CPW:
            if o[nb] is not None:
                wait_all(o[nb])
            g[nb] = gather(j + 1, nb)
        g[b].wait()
        o[b] = outcopy(j, b)
    wait_all(o[0])
    wait_all(o[1])


_gather = pl.kernel(
    _body,
    out_type=jax.ShapeDtypeStruct((BATCH, HIST, EMBED_DIM), jnp.float32),
    mesh=_mesh,
    scratch_types=[
        pltpu.VMEM((CPW, CHUNK), jnp.int32),
        pltpu.VMEM((CHUNK, EMBED_DIM), jnp.float32),
        pltpu.VMEM((CHUNK, EMBED_DIM), jnp.float32),
        pltpu.SemaphoreType.DMA,
        pltpu.SemaphoreType.DMA,
        pltpu.SemaphoreType.DMA,
        pltpu.SemaphoreType.DMA,
    ],
    compiler_params=pltpu.CompilerParams(use_tc_tiling_on_sc=False),
)


def kernel(input, weight):
    idx = input.reshape(NW, CPW, CHUNK).astype(jnp.int32)
    return _gather(idx, weight)
